# baseline (speedup 1.0000x reference)
import jax
import jax.numpy as jnp
from jax import lax
from jax.experimental import pallas as pl
from jax.experimental.pallas import tpu as pltpu
from jax.experimental.pallas import tpu_sc as plsc

N = 100000
EMB = 512
NF = 9
NC, NS = 2, 16
NW = NC * NS
CHUNK = 48
CHUNKS_PER_W = 23
LANES = 16

TC_BN = 512
NTC = 128 * TC_BN
NSC = N - NTC
NW_A = 29
WLEN_A = 1080
WLEN_B = 1048
FEAT_DIMS = (119, 4, 11, 12, 9, 5, 8, 2, 2)
KTOT = sum(FEAT_DIMS)
KP = 256
FEAT_OFF = tuple(sum(FEAT_DIMS[:i]) for i in range(NF))

A_ROWS = 119 * 8 * 2 * 2
B1_ROWS = 4 * 11 * 12
B2_ROWS = 9 * 5
B_ROWS = B1_ROWS * B2_ROWS
B_BLK = 8


def _outer(parts):
    acc = parts[0]
    for p in parts[1:]:
        acc = jnp.repeat(acc, p.shape[0], axis=0) + jnp.tile(p, (acc.shape[0], 1))
    return acc


def _build1_body(w0, w1, w2, w3, w4, w5, w6, w7, w8, a, b1, b2):
    a[...] = _outer([w0[...], w6[...], w7[...], w8[...]])
    b1[...] = _outer([w1[...], w2[...], w3[...]])
    b2[...] = _outer([w4[...], w5[...]])


_build1 = pl.pallas_call(
    _build1_body,
    out_shape=(
        jax.ShapeDtypeStruct((A_ROWS, EMB), jnp.float32),
        jax.ShapeDtypeStruct((B1_ROWS, EMB), jnp.float32),
        jax.ShapeDtypeStruct((B2_ROWS, EMB), jnp.float32),
    ),
)


def _build2_body(b1_blk, b2, b):
    b[...] = jnp.repeat(b1_blk[...], B2_ROWS, axis=0) + jnp.tile(b2[...], (B_BLK, 1))


_build2 = pl.pallas_call(
    _build2_body,
    grid=(B1_ROWS // B_BLK,),
    in_specs=[
        pl.BlockSpec((B_BLK, EMB), lambda i: (i, 0)),
        pl.BlockSpec((B2_ROWS, EMB), lambda i: (0, 0)),
    ],
    out_specs=pl.BlockSpec((B_BLK * B2_ROWS, EMB), lambda i: (i, 0)),
    out_shape=jax.ShapeDtypeStruct((B_ROWS, EMB), jnp.float32),
)


NTC_BLOCKS = NTC // TC_BN


def _tc_body(xt_ref, w_ref, sc_ref, o_ref):
    del sc_ref
    xb = xt_ref[...]
    iota0 = lax.broadcasted_iota(jnp.int32, (KP, TC_BN), 0)
    oh = (iota0 == xb[0:1, :] + FEAT_OFF[0]).astype(jnp.float32)
    for f in range(1, NF):
        oh += (iota0 == xb[f:f + 1, :] + FEAT_OFF[f]).astype(jnp.float32)
    o_ref[...] = lax.dot_general(oh.astype(jnp.bfloat16), w_ref[...],
                                 (((0,), (0,)), ((), ())),
                                 preferred_element_type=jnp.float32)


_tc_call = pl.pallas_call(
    _tc_body,
    grid=(NTC_BLOCKS,),
    in_specs=[
        pl.BlockSpec((NF, TC_BN), lambda i: (0, i)),
        pl.BlockSpec((KP, EMB), lambda i: (0, 0)),
        pl.BlockSpec(memory_space=pl.ANY),
    ],
    out_specs=pl.BlockSpec((TC_BN, EMB), lambda i: (i, 0)),
    out_shape=jax.ShapeDtypeStruct((N, EMB), jnp.float32),
    input_output_aliases={2: 0},
)


def _wbase(w):
    return w * WLEN_A - max(w - NW_A, 0) * (WLEN_A - WLEN_B)


def _wlen(w):
    return WLEN_A if w < NW_A else WLEN_B


CHUNK_STARTS = tuple(
    min(_wbase(w) + k * CHUNK, _wbase(w) + _wlen(w) - CHUNK)
    for w in range(NW) for k in range(CHUNKS_PER_W)
)
NCHUNKS = NW * CHUNKS_PER_W


def _sc_body(xr, ga, gb, out,
             idx_a, idx_b, code_a, code_b,
             a0, a1, b0, b1, sem_a, sem_b):
    wid = lax.axis_index("s") * NC + lax.axis_index("c")
    base = NTC + wid * WLEN_A - jnp.maximum(wid - NW_A, 0) * (WLEN_A - WLEN_B)
    wlen = jnp.where(wid < NW_A, WLEN_A, WLEN_B)
    last = base + wlen - CHUNK
    sets = ((idx_a, code_a, (a0, a1), sem_a),
            (idx_b, code_b, (b0, b1), sem_b))

    def start_of(k):
        return jnp.minimum(base + k * CHUNK, last)

    def prep(k, st):
        idx_v, code_v, bufs, sem = st
        c = wid * CHUNKS_PER_W + k
        pltpu.sync_copy(xr.at[c], idx_v)
        for j in range(CHUNK // LANES):
            s = pl.ds(j * LANES, LANES)
            code_v[0, s] = ((idx_v[0, s] * 32 + idx_v[6, s] * 4)
                            + (idx_v[7, s] * 2 + idx_v[8, s]))
            code_v[1, s] = ((idx_v[1, s] * 5940 + idx_v[2, s] * 540)
                            + (idx_v[3, s] * 45 + idx_v[4, s] * 5 + idx_v[5, s]))
        pltpu.async_copy(ga.at[code_v.at[0]], bufs[0], sem)
        pltpu.async_copy(gb.at[code_v.at[1]], bufs[1], sem)

    def finish(k, st):
        _, _, bufs, sem = st
        pltpu.make_async_copy(ga.at[pl.ds(0, CHUNK)], bufs[0], sem).wait()
        pltpu.make_async_copy(gb.at[pl.ds(0, CHUNK)], bufs[1], sem).wait()
        t0, t1 = bufs

        @pl.loop(0, CHUNK)
        def _row(r):
            for cc in range(EMB // LANES):
                s = pl.ds(cc * LANES, LANES)
                t0[r, s] = t0[r, s] + t1[r, s]

        pltpu.sync_copy(t0, out.at[pl.ds(start_of(k), CHUNK)])

    prep(0, sets[0])

    @pl.loop(0, CHUNKS_PER_W - 1, step=2)
    def _pipe(k):
        prep(k + 1, sets[1])
        finish(k, sets[0])
        prep(k + 2, sets[0])
        finish(k + 1, sets[1])

    finish(CHUNKS_PER_W - 1, sets[0])


_mesh = plsc.VectorSubcoreMesh(core_axis_name="c", subcore_axis_name="s",
                               num_cores=NC, num_subcores=NS)

_sc_call = pl.kernel(
    _sc_body,
    out_type=jax.ShapeDtypeStruct((N, EMB), jnp.float32),
    mesh=_mesh,
    scratch_types=[
        pltpu.VMEM((NF, CHUNK), jnp.int32),
        pltpu.VMEM((NF, CHUNK), jnp.int32),
        pltpu.VMEM((2, CHUNK), jnp.int32),
        pltpu.VMEM((2, CHUNK), jnp.int32),
        pltpu.VMEM((CHUNK, EMB), jnp.float32),
        pltpu.VMEM((CHUNK, EMB), jnp.float32),
        pltpu.VMEM((CHUNK, EMB), jnp.float32),
        pltpu.VMEM((CHUNK, EMB), jnp.float32),
        pltpu.SemaphoreType.DMA,
        pltpu.SemaphoreType.DMA,
    ],
)


def kernel(x, W0, W1, W2, W3, W4, W5, W6, W7, W8):
    a, b1, b2 = _build1(W0, W1, W2, W3, W4, W5, W6, W7, W8)
    b = _build2(b1, b2)
    starts = jnp.asarray(CHUNK_STARTS, dtype=jnp.int32)
    rows = (starts[:, None] + jnp.arange(CHUNK, dtype=jnp.int32)[None, :])
    xr = x[NTC:].T[:, rows.reshape(-1)]
    xr = xr.reshape(NF, NCHUNKS, CHUNK).transpose(1, 0, 2)
    out_sc = _sc_call(xr, a, b)
    wcat = jnp.concatenate([W0, W1, W2, W3, W4, W5, W6, W7, W8], axis=0)
    wcat = jnp.pad(wcat, ((0, KP - KTOT), (0, 0))).astype(jnp.bfloat16)
    return _tc_call(x[:NTC].T, wcat, out_sc)


# speedup vs baseline: 1.0158x; 1.0158x over previous
import jax
import jax.numpy as jnp
from jax import lax
from jax.experimental import pallas as pl
from jax.experimental.pallas import tpu as pltpu
from jax.experimental.pallas import tpu_sc as plsc

N = 100000
EMB = 512
NF = 9
NC, NS = 2, 16
NW = NC * NS
CHUNK = 48
CHUNKS_PER_W = 19
LANES = 16

TC_BN = 512
NTC = 144 * TC_BN
NSC = N - NTC
NW_A = 29
WLEN_A = 824
WLEN_B = 792
FEAT_DIMS = (119, 4, 11, 12, 9, 5, 8, 2, 2)
KTOT = sum(FEAT_DIMS)
KP = 256
FEAT_OFF = tuple(sum(FEAT_DIMS[:i]) for i in range(NF))

A_ROWS = 119 * 8 * 2 * 2
B1_ROWS = 4 * 11 * 12
B2_ROWS = 9 * 5
B_ROWS = B1_ROWS * B2_ROWS
B_BLK = 8


def _outer(parts):
    acc = parts[0]
    for p in parts[1:]:
        acc = jnp.repeat(acc, p.shape[0], axis=0) + jnp.tile(p, (acc.shape[0], 1))
    return acc


def _build1_body(w0, w1, w2, w3, w4, w5, w6, w7, w8, a, b1, b2):
    a[...] = _outer([w0[...], w6[...], w7[...], w8[...]])
    b1[...] = _outer([w1[...], w2[...], w3[...]])
    b2[...] = _outer([w4[...], w5[...]])


_build1 = pl.pallas_call(
    _build1_body,
    out_shape=(
        jax.ShapeDtypeStruct((A_ROWS, EMB), jnp.float32),
        jax.ShapeDtypeStruct((B1_ROWS, EMB), jnp.float32),
        jax.ShapeDtypeStruct((B2_ROWS, EMB), jnp.float32),
    ),
)


def _build2_body(b1_blk, b2, b):
    b[...] = jnp.repeat(b1_blk[...], B2_ROWS, axis=0) + jnp.tile(b2[...], (B_BLK, 1))


_build2 = pl.pallas_call(
    _build2_body,
    grid=(B1_ROWS // B_BLK,),
    in_specs=[
        pl.BlockSpec((B_BLK, EMB), lambda i: (i, 0)),
        pl.BlockSpec((B2_ROWS, EMB), lambda i: (0, 0)),
    ],
    out_specs=pl.BlockSpec((B_BLK * B2_ROWS, EMB), lambda i: (i, 0)),
    out_shape=jax.ShapeDtypeStruct((B_ROWS, EMB), jnp.float32),
)


NTC_BLOCKS = NTC // TC_BN


def _tc_body(xt_ref, w_ref, sc_ref, o_ref):
    del sc_ref
    xb = xt_ref[...]
    iota0 = lax.broadcasted_iota(jnp.int32, (KP, TC_BN), 0)
    oh = (iota0 == xb[0:1, :] + FEAT_OFF[0]).astype(jnp.float32)
    for f in range(1, NF):
        oh += (iota0 == xb[f:f + 1, :] + FEAT_OFF[f]).astype(jnp.float32)
    o_ref[...] = lax.dot_general(oh.astype(jnp.bfloat16), w_ref[...],
                                 (((0,), (0,)), ((), ())),
                                 preferred_element_type=jnp.float32)


_tc_call = pl.pallas_call(
    _tc_body,
    grid=(NTC_BLOCKS,),
    in_specs=[
        pl.BlockSpec((NF, TC_BN), lambda i: (0, i)),
        pl.BlockSpec((KP, EMB), lambda i: (0, 0)),
        pl.BlockSpec(memory_space=pl.ANY),
    ],
    out_specs=pl.BlockSpec((TC_BN, EMB), lambda i: (i, 0)),
    out_shape=jax.ShapeDtypeStruct((N, EMB), jnp.float32),
    input_output_aliases={2: 0},
)


def _wbase(w):
    return w * WLEN_A - max(w - NW_A, 0) * (WLEN_A - WLEN_B)


def _wlen(w):
    return WLEN_A if w < NW_A else WLEN_B


CHUNK_STARTS = tuple(
    min(_wbase(w) + k * CHUNK, _wbase(w) + _wlen(w) - CHUNK)
    for w in range(NW) for k in range(CHUNKS_PER_W)
)
NCHUNKS = NW * CHUNKS_PER_W


def _sc_body(xr, ga, gb, out,
             idx_a, idx_b, code_a, code_b,
             a0, a1, b0, b1, sem_a, sem_b):
    wid = lax.axis_index("s") * NC + lax.axis_index("c")
    base = NTC + wid * WLEN_A - jnp.maximum(wid - NW_A, 0) * (WLEN_A - WLEN_B)
    wlen = jnp.where(wid < NW_A, WLEN_A, WLEN_B)
    last = base + wlen - CHUNK
    sets = ((idx_a, code_a, (a0, a1), sem_a),
            (idx_b, code_b, (b0, b1), sem_b))

    def start_of(k):
        return jnp.minimum(base + k * CHUNK, last)

    def prep(k, st):
        idx_v, code_v, bufs, sem = st
        c = wid * CHUNKS_PER_W + k
        pltpu.sync_copy(xr.at[c], idx_v)
        for j in range(CHUNK // LANES):
            s = pl.ds(j * LANES, LANES)
            code_v[0, s] = ((idx_v[0, s] * 32 + idx_v[6, s] * 4)
                            + (idx_v[7, s] * 2 + idx_v[8, s]))
            code_v[1, s] = ((idx_v[1, s] * 5940 + idx_v[2, s] * 540)
                            + (idx_v[3, s] * 45 + idx_v[4, s] * 5 + idx_v[5, s]))
        pltpu.async_copy(ga.at[code_v.at[0]], bufs[0], sem)
        pltpu.async_copy(gb.at[code_v.at[1]], bufs[1], sem)

    def finish(k, st):
        _, _, bufs, sem = st
        pltpu.make_async_copy(ga.at[pl.ds(0, CHUNK)], bufs[0], sem).wait()
        pltpu.make_async_copy(gb.at[pl.ds(0, CHUNK)], bufs[1], sem).wait()
        t0, t1 = bufs

        @pl.loop(0, CHUNK)
        def _row(r):
            for cc in range(EMB // LANES):
                s = pl.ds(cc * LANES, LANES)
                t0[r, s] = t0[r, s] + t1[r, s]

        pltpu.sync_copy(t0, out.at[pl.ds(start_of(k), CHUNK)])

    prep(0, sets[0])

    @pl.loop(0, CHUNKS_PER_W - 1, step=2)
    def _pipe(k):
        prep(k + 1, sets[1])
        finish(k, sets[0])
        prep(k + 2, sets[0])
        finish(k + 1, sets[1])

    finish(CHUNKS_PER_W - 1, sets[0])


_mesh = plsc.VectorSubcoreMesh(core_axis_name="c", subcore_axis_name="s",
                               num_cores=NC, num_subcores=NS)

_sc_call = pl.kernel(
    _sc_body,
    out_type=jax.ShapeDtypeStruct((N, EMB), jnp.float32),
    mesh=_mesh,
    scratch_types=[
        pltpu.VMEM((NF, CHUNK), jnp.int32),
        pltpu.VMEM((NF, CHUNK), jnp.int32),
        pltpu.VMEM((2, CHUNK), jnp.int32),
        pltpu.VMEM((2, CHUNK), jnp.int32),
        pltpu.VMEM((CHUNK, EMB), jnp.float32),
        pltpu.VMEM((CHUNK, EMB), jnp.float32),
        pltpu.VMEM((CHUNK, EMB), jnp.float32),
        pltpu.VMEM((CHUNK, EMB), jnp.float32),
        pltpu.SemaphoreType.DMA,
        pltpu.SemaphoreType.DMA,
    ],
)


def kernel(x, W0, W1, W2, W3, W4, W5, W6, W7, W8):
    a, b1, b2 = _build1(W0, W1, W2, W3, W4, W5, W6, W7, W8)
    b = _build2(b1, b2)
    starts = jnp.asarray(CHUNK_STARTS, dtype=jnp.int32)
    rows = (starts[:, None] + jnp.arange(CHUNK, dtype=jnp.int32)[None, :])
    xr = x[NTC:].T[:, rows.reshape(-1)]
    xr = xr.reshape(NF, NCHUNKS, CHUNK).transpose(1, 0, 2)
    out_sc = _sc_call(xr, a, b)
    wcat = jnp.concatenate([W0, W1, W2, W3, W4, W5, W6, W7, W8], axis=0)
    wcat = jnp.pad(wcat, ((0, KP - KTOT), (0, 0))).astype(jnp.bfloat16)
    return _tc_call(x[:NTC].T, wcat, out_sc)
